# baseline (device time: 17411 ns/iter reference)
import jax
import jax.numpy as jnp
from jax import lax
from jax.experimental import pallas as pl
from jax.experimental.pallas import tpu as pltpu

N_DEV = 8
CHUNK = 512 // N_DEV
KT = 4


def kernel(dy, W):
    m, k = dy.shape
    d = W.shape[0]
    KTS = k // KT
    NSPL = 4
    HALF = d // NSPL

    def body(dy_hbm, w_hbm, out_ref, dy_v, w_v, part_ref, red_ref,
             rs_buf, ag_buf, in_sems,
             rs_send, rs_recv, ag_send, ag_recv):
        my = lax.axis_index("i")

        cps = []
        for t in range(KT):
            c_dy = pltpu.make_async_copy(
                dy_hbm.at[:, pl.ds(t * KTS, KTS)], dy_v.at[t],
                in_sems.at[2 * t],
            )
            c_w = pltpu.make_async_copy(
                w_hbm.at[:, pl.ds(t * KTS, KTS)], w_v.at[t],
                in_sems.at[2 * t + 1],
            )
            c_dy.start()
            c_w.start()
            cps.append((c_dy, c_w))

        barrier_sem = pltpu.get_barrier_semaphore()
        for r in range(1, N_DEV):
            pl.semaphore_signal(
                barrier_sem, inc=1,
                device_id=((my + r) % N_DEV,),
                device_id_type=pl.DeviceIdType.MESH,
            )

        partial = None
        for t in range(KT):
            cps[t][0].wait()
            cps[t][1].wait()
            p = lax.dot_general(
                dy_v[t, :, :], w_v[t, :, :],
                (((1,), (1,)), ((), ())),
                preferred_element_type=jnp.float32,
            )
            partial = p if partial is None else partial + p
        part_ref[:, :, :] = partial.astype(jnp.bfloat16).reshape(
            N_DEV, CHUNK, d
        )

        pl.semaphore_wait(barrier_sem, N_DEV - 1)

        rs_rdmas = []
        for h in range(NSPL):
            for r in range(1, N_DEV):
                dst = (my + r) % N_DEV
                rdma = pltpu.make_async_remote_copy(
                    src_ref=part_ref.at[dst, :, pl.ds(h * HALF, HALF)],
                    dst_ref=rs_buf.at[h, N_DEV - 1 - r],
                    send_sem=rs_send.at[h, r - 1],
                    recv_sem=rs_recv.at[h, N_DEV - 1 - r],
                    device_id=(dst,),
                    device_id_type=pl.DeviceIdType.MESH,
                )
                rdma.start()
                rs_rdmas.append(rdma)

        ag_rdmas = []
        red_halves = []
        for h in range(NSPL):
            for s in range(N_DEV - 1):
                pltpu.make_async_remote_copy(
                    src_ref=rs_buf.at[h, s], dst_ref=rs_buf.at[h, s],
                    send_sem=rs_send.at[h, s], recv_sem=rs_recv.at[h, s],
                    device_id=(my,), device_id_type=pl.DeviceIdType.MESH,
                ).wait_recv()
            terms = [
                part_ref[my, :, pl.ds(h * HALF, HALF)].astype(jnp.float32)
            ] + [
                rs_buf[h, s, :, :].astype(jnp.float32)
                for s in range(N_DEV - 1)
            ]
            while len(terms) > 1:
                terms = [
                    terms[i] + terms[i + 1]
                    if i + 1 < len(terms) else terms[i]
                    for i in range(0, len(terms), 2)
                ]
            red = terms[0]
            red_halves.append(red)
            red_ref[h, :, :] = red.astype(jnp.bfloat16)

            for r in range(1, N_DEV):
                dst = (my + r) % N_DEV
                rdma = pltpu.make_async_remote_copy(
                    src_ref=red_ref.at[h],
                    dst_ref=ag_buf.at[h, N_DEV - 1 - r],
                    send_sem=ag_send.at[h, r - 1],
                    recv_sem=ag_recv.at[h, N_DEV - 1 - r],
                    device_id=(dst,),
                    device_id_type=pl.DeviceIdType.MESH,
                )
                rdma.start()
                ag_rdmas.append(rdma)

        for h in range(NSPL):
            out_ref[pl.ds(my * CHUNK, CHUNK), pl.ds(h * HALF, HALF)] = (
                red_halves[h]
            )

        for h in range(NSPL):
            for s in range(N_DEV - 1):
                pltpu.make_async_remote_copy(
                    src_ref=ag_buf.at[h, s], dst_ref=ag_buf.at[h, s],
                    send_sem=ag_send.at[h, s], recv_sem=ag_recv.at[h, s],
                    device_id=(my,), device_id_type=pl.DeviceIdType.MESH,
                ).wait_recv()
                origin = (my + s + 1) % N_DEV
                out_ref[
                    pl.ds(origin * CHUNK, CHUNK), pl.ds(h * HALF, HALF)
                ] = ag_buf[h, s, :, :].astype(jnp.float32)

        for rdma in rs_rdmas + ag_rdmas:
            rdma.wait_send()

    return pl.pallas_call(
        body,
        out_shape=jax.ShapeDtypeStruct((m, d), jnp.float32),
        in_specs=[
            pl.BlockSpec(memory_space=pltpu.MemorySpace.HBM),
            pl.BlockSpec(memory_space=pltpu.MemorySpace.HBM),
        ],
        out_specs=pl.BlockSpec(memory_space=pltpu.VMEM),
        scratch_shapes=[
            pltpu.VMEM((KT, m, k // KT), jnp.float32),
            pltpu.VMEM((KT, d, k // KT), jnp.float32),
            pltpu.VMEM((N_DEV, CHUNK, d), jnp.bfloat16),
            pltpu.VMEM((4, CHUNK, d // 4), jnp.bfloat16),
            pltpu.VMEM((4, N_DEV - 1, CHUNK, d // 4), jnp.bfloat16),
            pltpu.VMEM((4, N_DEV - 1, CHUNK, d // 4), jnp.bfloat16),
            pltpu.SemaphoreType.DMA((2 * KT,)),
            pltpu.SemaphoreType.DMA((4, N_DEV - 1)),
            pltpu.SemaphoreType.DMA((4, N_DEV - 1)),
            pltpu.SemaphoreType.DMA((4, N_DEV - 1)),
            pltpu.SemaphoreType.DMA((4, N_DEV - 1)),
        ],
        compiler_params=pltpu.CompilerParams(collective_id=0),
    )(
        pltpu.with_memory_space_constraint(dy, pltpu.MemorySpace.HBM),
        pltpu.with_memory_space_constraint(W, pltpu.MemorySpace.HBM),
    )


# device time: 17221 ns/iter; 1.0110x vs baseline; 1.0110x over previous
import jax
import jax.numpy as jnp
from jax import lax
from jax.experimental import pallas as pl
from jax.experimental.pallas import tpu as pltpu

N_DEV = 8
CHUNK = 512 // N_DEV
KT = 8


def kernel(dy, W):
    m, k = dy.shape
    d = W.shape[0]
    KTS = k // KT
    HALF = d // 2

    def body(dy_hbm, w_hbm, out_ref, dy_v, w_v, part_ref, red_ref,
             rs_buf, ag_buf, in_sems,
             rs_send, rs_recv, ag_send, ag_recv):
        my = lax.axis_index("i")

        cps = []
        for t in range(KT):
            c_dy = pltpu.make_async_copy(
                dy_hbm.at[:, pl.ds(t * KTS, KTS)], dy_v.at[t],
                in_sems.at[2 * t],
            )
            c_w = pltpu.make_async_copy(
                w_hbm.at[:, pl.ds(t * KTS, KTS)], w_v.at[t],
                in_sems.at[2 * t + 1],
            )
            c_dy.start()
            c_w.start()
            cps.append((c_dy, c_w))

        barrier_sem = pltpu.get_barrier_semaphore()
        for r in range(1, N_DEV):
            pl.semaphore_signal(
                barrier_sem, inc=1,
                device_id=((my + r) % N_DEV,),
                device_id_type=pl.DeviceIdType.MESH,
            )

        partial = None
        for t in range(KT):
            cps[t][0].wait()
            cps[t][1].wait()
            p = lax.dot_general(
                dy_v[t, :, :], w_v[t, :, :],
                (((1,), (1,)), ((), ())),
                preferred_element_type=jnp.float32,
            )
            partial = p if partial is None else partial + p
        part_ref[:, :, :] = partial.astype(jnp.bfloat16).reshape(
            N_DEV, CHUNK, d
        )

        pl.semaphore_wait(barrier_sem, N_DEV - 1)

        rs_rdmas = []
        for h in range(2):
            for r in range(1, N_DEV):
                dst = (my + r) % N_DEV
                rdma = pltpu.make_async_remote_copy(
                    src_ref=part_ref.at[dst, :, pl.ds(h * HALF, HALF)],
                    dst_ref=rs_buf.at[h, N_DEV - 1 - r],
                    send_sem=rs_send.at[h, r - 1],
                    recv_sem=rs_recv.at[h, N_DEV - 1 - r],
                    device_id=(dst,),
                    device_id_type=pl.DeviceIdType.MESH,
                )
                rdma.start()
                rs_rdmas.append(rdma)

        ag_rdmas = []
        red_halves = []
        for h in range(2):
            for s in range(N_DEV - 1):
                pltpu.make_async_remote_copy(
                    src_ref=rs_buf.at[h, s], dst_ref=rs_buf.at[h, s],
                    send_sem=rs_send.at[h, s], recv_sem=rs_recv.at[h, s],
                    device_id=(my,), device_id_type=pl.DeviceIdType.MESH,
                ).wait_recv()
            terms = [
                part_ref[my, :, pl.ds(h * HALF, HALF)].astype(jnp.float32)
            ] + [
                rs_buf[h, s, :, :].astype(jnp.float32)
                for s in range(N_DEV - 1)
            ]
            while len(terms) > 1:
                terms = [
                    terms[i] + terms[i + 1]
                    if i + 1 < len(terms) else terms[i]
                    for i in range(0, len(terms), 2)
                ]
            red = terms[0]
            red_halves.append(red)
            red_ref[h, :, :] = red.astype(jnp.bfloat16)

            for r in range(1, N_DEV):
                dst = (my + r) % N_DEV
                rdma = pltpu.make_async_remote_copy(
                    src_ref=red_ref.at[h],
                    dst_ref=ag_buf.at[h, N_DEV - 1 - r],
                    send_sem=ag_send.at[h, r - 1],
                    recv_sem=ag_recv.at[h, N_DEV - 1 - r],
                    device_id=(dst,),
                    device_id_type=pl.DeviceIdType.MESH,
                )
                rdma.start()
                ag_rdmas.append(rdma)

        for h in range(2):
            out_ref[pl.ds(my * CHUNK, CHUNK), pl.ds(h * HALF, HALF)] = (
                red_halves[h]
            )

        for h in range(2):
            for s in range(N_DEV - 1):
                pltpu.make_async_remote_copy(
                    src_ref=ag_buf.at[h, s], dst_ref=ag_buf.at[h, s],
                    send_sem=ag_send.at[h, s], recv_sem=ag_recv.at[h, s],
                    device_id=(my,), device_id_type=pl.DeviceIdType.MESH,
                ).wait_recv()
                origin = (my + s + 1) % N_DEV
                out_ref[
                    pl.ds(origin * CHUNK, CHUNK), pl.ds(h * HALF, HALF)
                ] = ag_buf[h, s, :, :].astype(jnp.float32)

        for rdma in rs_rdmas + ag_rdmas:
            rdma.wait_send()

    return pl.pallas_call(
        body,
        out_shape=jax.ShapeDtypeStruct((m, d), jnp.float32),
        in_specs=[
            pl.BlockSpec(memory_space=pltpu.MemorySpace.HBM),
            pl.BlockSpec(memory_space=pltpu.MemorySpace.HBM),
        ],
        out_specs=pl.BlockSpec(memory_space=pltpu.VMEM),
        scratch_shapes=[
            pltpu.VMEM((KT, m, k // KT), jnp.float32),
            pltpu.VMEM((KT, d, k // KT), jnp.float32),
            pltpu.VMEM((N_DEV, CHUNK, d), jnp.bfloat16),
            pltpu.VMEM((2, CHUNK, d // 2), jnp.bfloat16),
            pltpu.VMEM((2, N_DEV - 1, CHUNK, d // 2), jnp.bfloat16),
            pltpu.VMEM((2, N_DEV - 1, CHUNK, d // 2), jnp.bfloat16),
            pltpu.SemaphoreType.DMA((2 * KT,)),
            pltpu.SemaphoreType.DMA((2, N_DEV - 1)),
            pltpu.SemaphoreType.DMA((2, N_DEV - 1)),
            pltpu.SemaphoreType.DMA((2, N_DEV - 1)),
            pltpu.SemaphoreType.DMA((2, N_DEV - 1)),
        ],
        compiler_params=pltpu.CompilerParams(collective_id=0),
    )(
        pltpu.with_memory_space_constraint(dy, pltpu.MemorySpace.HBM),
        pltpu.with_memory_space_constraint(W, pltpu.MemorySpace.HBM),
    )
